# baseline (speedup 1.0000x reference)
import functools

import jax
import jax.numpy as jnp
from jax.experimental import pallas as pl
from jax.experimental.pallas import tpu as pltpu

_TAU = 2.0
_GAMMA = 0.7
_IOU_THR = 0.5

_TI = 256
_TJ = 640


def _stats_kernel(tb_ref, sb_ref, tl_ref, tm_ref, sl_ref,
                  fa_ref, fb_ref, cp_ref, lse_ref):
    tx1 = tb_ref[0, :, 0:1]
    ty1 = tb_ref[0, :, 1:2]
    tx2 = tb_ref[0, :, 2:3]
    ty2 = tb_ref[0, :, 3:4]
    sx1 = sb_ref[0, :, 0:1]
    sy1 = sb_ref[0, :, 1:2]
    sx2 = sb_ref[0, :, 2:3]
    sy2 = sb_ref[0, :, 3:4]
    area_t = (tx2 - tx1) * (ty2 - ty1)
    area_s = (sx2 - sx1) * (sy2 - sy1)
    wx = jnp.maximum(jnp.minimum(tx2, sx2) - jnp.maximum(tx1, sx1), 0.0)
    wy = jnp.maximum(jnp.minimum(ty2, sy2) - jnp.maximum(ty1, sy1), 0.0)
    inter = wx * wy
    pass0 = inter - area_s * (1.0 / 3.0) >= area_t * (1.0 / 3.0)
    tmv = tm_ref[0] > 0.5
    fa_ref[0] = jnp.where(tmv & jnp.logical_not(pass0), 1.0, 0.0)
    cp = jnp.sum(jnp.where(tmv & pass0, 1.0, 0.0))
    cp_ref[...] = jnp.full(cp_ref.shape, cp, jnp.float32)

    zt = tl_ref[0] * (1.0 / _TAU)
    mt = jnp.max(zt, axis=1, keepdims=True)
    st = jnp.sum(jnp.exp(zt - mt), axis=1, keepdims=True)
    fb_ref[0] = jnp.where(tmv & (st < (1.0 / _GAMMA) * (1.0 + 1e-5)),
                          1.0, 0.0)

    zs = sl_ref[0] * (1.0 / _TAU)
    ms = jnp.max(zs, axis=1, keepdims=True)
    lse_ref[0] = ms + jnp.log(jnp.sum(jnp.exp(zs - ms), axis=1,
                                      keepdims=True))


def _scan_kernel(idxa_s, na_s, idxb_s, nb_s,
                 tb_ref, tl_ref, sbt_ref, sl_ref, lse_ref,
                 cnt_ref, sum_ref, tbs, tls, *, mp, nj):
    i = pl.program_id(0)
    t = pl.program_id(1)
    base = t * _TI

    @pl.when(t == 0)
    def _():
        cnt_ref[...] = jnp.zeros_like(cnt_ref)
        sum_ref[...] = jnp.zeros_like(sum_ref)

    na = na_s[i]
    nb = nb_s[i]
    rows_a = jnp.clip(na - base, 0, _TI)
    rows_b = jnp.clip(nb - base, 0, _TI)

    def stile(jt):
        j0 = jt * _TJ
        sx1 = sbt_ref[0, 0:1, j0:j0 + _TJ]
        sy1 = sbt_ref[0, 1:2, j0:j0 + _TJ]
        sx2 = sbt_ref[0, 2:3, j0:j0 + _TJ]
        sy2 = sbt_ref[0, 3:4, j0:j0 + _TJ]
        return sx1, sy1, sx2, sy2, (sx2 - sx1) * (sy2 - sy1)

    def gather_a(r, c):
        g = idxa_s[i * mp + base + r]
        tbs[pl.ds(r, 1), :] = tb_ref[0, pl.ds(g, 1), :]
        return c

    jax.lax.fori_loop(0, rows_a, gather_a, 0)

    def scan_a():
        tx1 = tbs[:, 0:1]
        ty1 = tbs[:, 1:2]
        tx2 = tbs[:, 2:3]
        ty2 = tbs[:, 3:4]
        area_t3 = (tx2 - tx1) * (ty2 - ty1) * (1.0 / 3.0)
        hmax = jnp.full((_TI, 1), -jnp.inf, jnp.float32)
        for jt in range(nj):
            sx1, sy1, sx2, sy2, area_s = stile(jt)
            wx = jnp.maximum(jnp.minimum(tx2, sx2) - jnp.maximum(tx1, sx1),
                             0.0)
            wy = jnp.maximum(jnp.minimum(ty2, sy2) - jnp.maximum(ty1, sy1),
                             0.0)
            h = wx * wy - area_s * (1.0 / 3.0)
            hmax = jnp.maximum(hmax, jnp.max(h, axis=1, keepdims=True))
        valid = jax.lax.broadcasted_iota(jnp.int32, (_TI, 1), 0) < rows_a
        kept = (hmax >= area_t3) & valid
        return jnp.sum(jnp.where(kept, 1.0, 0.0))

    cnt_add = jax.lax.cond(rows_a > 0, scan_a, lambda: 0.0)

    def gather_b(r, c):
        g = idxb_s[i * mp + base + r]
        tbs[pl.ds(r, 1), :] = tb_ref[0, pl.ds(g, 1), :]
        tls[pl.ds(r, 1), :] = tl_ref[0, pl.ds(g, 1), :]
        return c

    jax.lax.fori_loop(0, rows_b, gather_b, 0)

    def scan_b():
        tx1 = tbs[:, 0:1]
        ty1 = tbs[:, 1:2]
        tx2 = tbs[:, 2:3]
        ty2 = tbs[:, 3:4]
        area_t = (tx2 - tx1) * (ty2 - ty1)

        zt = tls[...] * (1.0 / _TAU)
        mt = jnp.max(zt, axis=1, keepdims=True)
        et = jnp.exp(zt - mt)
        st = jnp.sum(et, axis=1, keepdims=True)
        lse_t = mt + jnp.log(st)
        p_t = et / st
        ent = jnp.sum(p_t * zt, axis=1, keepdims=True) - lse_t
        conf = jnp.max(p_t, axis=1, keepdims=True)
        w = jnp.clip((conf - _GAMMA) / (1.0 - _GAMMA), 0.0, 1.0)
        pts = p_t * (1.0 / _TAU)

        def iou_tile(jt):
            sx1, sy1, sx2, sy2, area_s = stile(jt)
            wx = jnp.maximum(jnp.minimum(tx2, sx2) - jnp.maximum(tx1, sx1),
                             0.0)
            wy = jnp.maximum(jnp.minimum(ty2, sy2) - jnp.maximum(ty1, sy1),
                             0.0)
            inter = wx * wy
            union = area_t + area_s - inter
            return inter / jnp.maximum(union, 1e-12), inter, area_s

        best = jnp.full((_TI, 1), -jnp.inf, jnp.float32)
        hmax = jnp.full((_TI, 1), -jnp.inf, jnp.float32)
        for jt in range(nj):
            iou, inter, area_s = iou_tile(jt)
            h = inter - area_s * (1.0 / 3.0)
            hmax = jnp.maximum(hmax, jnp.max(h, axis=1, keepdims=True))
            best = jnp.maximum(best, jnp.max(iou, axis=1, keepdims=True))

        qb = jnp.full((_TI, 1), -jnp.inf, jnp.float32)
        for jt in range(nj):
            j0 = jt * _TJ
            g = jax.lax.dot_general(
                pts, sl_ref[0, j0:j0 + _TJ, :],
                dimension_numbers=(((1,), (1,)), ((), ())),
                preferred_element_type=jnp.float32)
            q = lse_ref[0, 0:1, j0:j0 + _TJ] - g
            iou, _, _ = iou_tile(jt)
            qsel = jnp.max(jnp.where(iou == best, q, -jnp.inf), axis=1,
                           keepdims=True)
            qb = jnp.maximum(qb, qsel)

        valid = jax.lax.broadcasted_iota(jnp.int32, (_TI, 1), 0) < rows_b
        kept = (hmax >= area_t * (1.0 / 3.0)) & valid
        kl = ent + qb
        terms = w * (_TAU * _TAU) * kl
        return jnp.sum(jnp.where(kept, terms, 0.0))

    sum_add = jax.lax.cond(rows_b > 0, scan_b, lambda: 0.0)

    cnt_ref[...] += jnp.full(cnt_ref.shape, cnt_add, jnp.float32)
    sum_ref[...] += jnp.full(sum_ref.shape, sum_add, jnp.float32)


def kernel(t_boxes, t_logits, t_valid, s_boxes, s_logits, s_valid):
    B, M, C = t_logits.shape
    dt = jnp.float32
    step = 1280
    MP = ((M + step - 1) // step) * step
    pad = MP - M

    tbp = jnp.pad(t_boxes.astype(dt), ((0, 0), (0, pad), (0, 0)))
    sbp = jnp.pad(s_boxes.astype(dt), ((0, 0), (0, pad), (0, 0)))
    tlp = jnp.pad(t_logits.astype(dt), ((0, 0), (0, pad), (0, 0)))
    tmf = jnp.pad(t_valid.astype(dt), ((0, 0), (0, pad)))[..., None]
    sbt = sbp.transpose(0, 2, 1)
    slp = jnp.pad(s_logits.astype(dt), ((0, 0), (0, pad), (0, 0)))

    fa, fb, cp, lse_col = pl.pallas_call(
        _stats_kernel,
        grid=(B,),
        in_specs=[
            pl.BlockSpec((1, MP, 4), lambda i: (i, 0, 0)),
            pl.BlockSpec((1, MP, 4), lambda i: (i, 0, 0)),
            pl.BlockSpec((1, MP, C), lambda i: (i, 0, 0)),
            pl.BlockSpec((1, MP, 1), lambda i: (i, 0, 0)),
            pl.BlockSpec((1, MP, C), lambda i: (i, 0, 0)),
        ],
        out_specs=[
            pl.BlockSpec((1, MP, 1), lambda i: (i, 0, 0)),
            pl.BlockSpec((1, MP, 1), lambda i: (i, 0, 0)),
            pl.BlockSpec((1, 8, 128), lambda i: (i, 0, 0)),
            pl.BlockSpec((1, MP, 1), lambda i: (i, 0, 0)),
        ],
        out_shape=[
            jax.ShapeDtypeStruct((B, MP, 1), dt),
            jax.ShapeDtypeStruct((B, MP, 1), dt),
            jax.ShapeDtypeStruct((B, 8, 128), dt),
            jax.ShapeDtypeStruct((B, MP, 1), dt),
        ],
        compiler_params=pltpu.CompilerParams(
            dimension_semantics=("parallel",)),
    )(tbp, sbp, tlp, tmf, slp)
    lse = lse_col.transpose(0, 2, 1)

    fa2 = fa[:, :, 0]
    fb2 = fb[:, :, 0]
    idxa = jnp.argsort(-fa2, axis=1).astype(jnp.int32).reshape(-1)
    idxb = jnp.argsort(-fb2, axis=1).astype(jnp.int32).reshape(-1)
    na = jnp.sum(fa2, axis=1).astype(jnp.int32)
    nb = jnp.sum(fb2, axis=1).astype(jnp.int32)

    nt = MP // _TI
    nj = MP // _TJ
    grid_spec = pltpu.PrefetchScalarGridSpec(
        num_scalar_prefetch=4,
        grid=(B, nt),
        in_specs=[
            pl.BlockSpec((1, MP, 4), lambda i, t, *_: (i, 0, 0)),
            pl.BlockSpec((1, MP, C), lambda i, t, *_: (i, 0, 0)),
            pl.BlockSpec((1, 4, MP), lambda i, t, *_: (i, 0, 0)),
            pl.BlockSpec((1, MP, C), lambda i, t, *_: (i, 0, 0)),
            pl.BlockSpec((1, 1, MP), lambda i, t, *_: (i, 0, 0)),
        ],
        out_specs=[
            pl.BlockSpec((1, 8, 128), lambda i, t, *_: (i, 0, 0)),
            pl.BlockSpec((1, 8, 128), lambda i, t, *_: (i, 0, 0)),
        ],
        scratch_shapes=[
            pltpu.VMEM((_TI, 4), dt),
            pltpu.VMEM((_TI, C), dt),
        ],
    )
    cnts, sums = pl.pallas_call(
        functools.partial(_scan_kernel, mp=MP, nj=nj),
        grid_spec=grid_spec,
        out_shape=[
            jax.ShapeDtypeStruct((B, 8, 128), dt),
            jax.ShapeDtypeStruct((B, 8, 128), dt),
        ],
        compiler_params=pltpu.CompilerParams(
            dimension_semantics=("parallel", "arbitrary")),
    )(idxa, na, idxb, nb, tbp, tlp, sbt, slp, lse)

    s = sums[:, 0, 0]
    n = cp[:, 0, 0] + cnts[:, 0, 0]
    has = n > 0
    loss_i = jnp.where(has, s / jnp.maximum(n, 1.0), 0.0)
    loss_sum = jnp.sum(loss_i)
    denom = jnp.sum(has.astype(dt))
    return jnp.where(denom == 0, loss_sum, loss_sum / jnp.maximum(denom, 1.0))


# speedup vs baseline: 1.3029x; 1.3029x over previous
import functools

import jax
import jax.numpy as jnp
from jax.experimental import pallas as pl
from jax.experimental.pallas import tpu as pltpu

_TAU = 2.0
_GAMMA = 0.7
_IOU_THR = 0.5

_TI = 256
_TJ = 640


def _stiles(m):
    return [(j0, min(_TJ, m - j0)) for j0 in range(0, m, _TJ)]


def _stats_kernel(tb_ref, sb_ref, tl_ref, tm_ref, sl_ref,
                  fa_ref, fb_ref, cp_ref, lse_ref):
    tx1 = tb_ref[0, :, 0:1]
    ty1 = tb_ref[0, :, 1:2]
    tx2 = tb_ref[0, :, 2:3]
    ty2 = tb_ref[0, :, 3:4]
    sx1 = sb_ref[0, :, 0:1]
    sy1 = sb_ref[0, :, 1:2]
    sx2 = sb_ref[0, :, 2:3]
    sy2 = sb_ref[0, :, 3:4]
    area_t = (tx2 - tx1) * (ty2 - ty1)
    area_s = (sx2 - sx1) * (sy2 - sy1)
    wx = jnp.maximum(jnp.minimum(tx2, sx2) - jnp.maximum(tx1, sx1), 0.0)
    wy = jnp.maximum(jnp.minimum(ty2, sy2) - jnp.maximum(ty1, sy1), 0.0)
    inter = wx * wy
    pass0 = inter - area_s * (1.0 / 3.0) >= area_t * (1.0 / 3.0)
    tmv = tm_ref[0] > 0.5
    fa_ref[0] = jnp.where(tmv & jnp.logical_not(pass0), 1.0, 0.0)
    cp = jnp.sum(jnp.where(tmv & pass0, 1.0, 0.0))
    cp_ref[...] = jnp.full(cp_ref.shape, cp, jnp.float32)

    zt = tl_ref[0] * (1.0 / _TAU)
    mt = jnp.max(zt, axis=1, keepdims=True)
    st = jnp.sum(jnp.exp(zt - mt), axis=1, keepdims=True)
    fb_ref[0] = jnp.where(tmv & (st < (1.0 / _GAMMA) * (1.0 + 1e-5)),
                          1.0, 0.0)

    zs = sl_ref[0] * (1.0 / _TAU)
    ms = jnp.max(zs, axis=1, keepdims=True)
    lse_ref[0] = ms + jnp.log(jnp.sum(jnp.exp(zs - ms), axis=1,
                                      keepdims=True))


def _scan_kernel(idxa_s, na_s, idxb_s, nb_s,
                 tb_ref, tl_ref, sbt_ref, sl_ref, lse_ref,
                 cnt_ref, sum_ref, tbs, tls, *, m):
    i = pl.program_id(0)
    t = pl.program_id(1)
    base = t * _TI

    @pl.when(t == 0)
    def _():
        cnt_ref[...] = jnp.zeros_like(cnt_ref)
        sum_ref[...] = jnp.zeros_like(sum_ref)

    na = na_s[i]
    nb = nb_s[i]
    rows_a = jnp.clip(na - base, 0, _TI)
    rows_b = jnp.clip(nb - base, 0, _TI)

    def stile(j0, tj):
        sx1 = sbt_ref[0, 0:1, j0:j0 + tj]
        sy1 = sbt_ref[0, 1:2, j0:j0 + tj]
        sx2 = sbt_ref[0, 2:3, j0:j0 + tj]
        sy2 = sbt_ref[0, 3:4, j0:j0 + tj]
        return sx1, sy1, sx2, sy2, (sx2 - sx1) * (sy2 - sy1)

    def gather_a(r, c):
        g = idxa_s[i * m + base + r]
        tbs[pl.ds(r, 1), :] = tb_ref[0, pl.ds(g, 1), :]
        return c

    jax.lax.fori_loop(0, rows_a, gather_a, 0)

    def scan_a():
        tx1 = tbs[:, 0:1]
        ty1 = tbs[:, 1:2]
        tx2 = tbs[:, 2:3]
        ty2 = tbs[:, 3:4]
        area_t3 = (tx2 - tx1) * (ty2 - ty1) * (1.0 / 3.0)
        hmax = jnp.full((_TI, 1), -jnp.inf, jnp.float32)
        for j0, tj in _stiles(m):
            sx1, sy1, sx2, sy2, area_s = stile(j0, tj)
            wx = jnp.maximum(jnp.minimum(tx2, sx2) - jnp.maximum(tx1, sx1),
                             0.0)
            wy = jnp.maximum(jnp.minimum(ty2, sy2) - jnp.maximum(ty1, sy1),
                             0.0)
            h = wx * wy - area_s * (1.0 / 3.0)
            hmax = jnp.maximum(hmax, jnp.max(h, axis=1, keepdims=True))
        valid = jax.lax.broadcasted_iota(jnp.int32, (_TI, 1), 0) < rows_a
        kept = (hmax >= area_t3) & valid
        return jnp.sum(jnp.where(kept, 1.0, 0.0))

    cnt_add = jax.lax.cond(rows_a > 0, scan_a, lambda: 0.0)

    def gather_b(r, c):
        g = idxb_s[i * m + base + r]
        tbs[pl.ds(r, 1), :] = tb_ref[0, pl.ds(g, 1), :]
        tls[pl.ds(r, 1), :] = tl_ref[0, pl.ds(g, 1), :]
        return c

    jax.lax.fori_loop(0, rows_b, gather_b, 0)

    def scan_b():
        tx1 = tbs[:, 0:1]
        ty1 = tbs[:, 1:2]
        tx2 = tbs[:, 2:3]
        ty2 = tbs[:, 3:4]
        area_t = (tx2 - tx1) * (ty2 - ty1)

        zt = tls[...] * (1.0 / _TAU)
        mt = jnp.max(zt, axis=1, keepdims=True)
        et = jnp.exp(zt - mt)
        st = jnp.sum(et, axis=1, keepdims=True)
        lse_t = mt + jnp.log(st)
        p_t = et / st
        ent = jnp.sum(p_t * zt, axis=1, keepdims=True) - lse_t
        conf = jnp.max(p_t, axis=1, keepdims=True)
        w = jnp.clip((conf - _GAMMA) / (1.0 - _GAMMA), 0.0, 1.0)
        pts_ext = jnp.concatenate(
            [p_t * (-1.0 / _TAU), jnp.ones((_TI, 1), jnp.float32)], axis=1)

        def iou_tile(j0, tj):
            sx1, sy1, sx2, sy2, area_s = stile(j0, tj)
            wx = jnp.maximum(jnp.minimum(tx2, sx2) - jnp.maximum(tx1, sx1),
                             0.0)
            wy = jnp.maximum(jnp.minimum(ty2, sy2) - jnp.maximum(ty1, sy1),
                             0.0)
            inter = wx * wy
            union = area_t + area_s - inter
            return inter / jnp.maximum(union, 1e-12), inter, area_s

        best = jnp.full((_TI, 1), -jnp.inf, jnp.float32)
        hmax = jnp.full((_TI, 1), -jnp.inf, jnp.float32)
        for j0, tj in _stiles(m):
            iou, inter, area_s = iou_tile(j0, tj)
            h = inter - area_s * (1.0 / 3.0)
            hmax = jnp.maximum(hmax, jnp.max(h, axis=1, keepdims=True))
            best = jnp.maximum(best, jnp.max(iou, axis=1, keepdims=True))

        qb = jnp.full((_TI, 1), -jnp.inf, jnp.float32)
        for j0, tj in _stiles(m):
            sl_ext = jnp.concatenate(
                [sl_ref[0, j0:j0 + tj, :], lse_ref[0, j0:j0 + tj, :]],
                axis=1)
            q = jax.lax.dot_general(
                pts_ext, sl_ext,
                dimension_numbers=(((1,), (1,)), ((), ())),
                preferred_element_type=jnp.float32)
            iou, _, _ = iou_tile(j0, tj)
            qsel = jnp.max(jnp.where(iou == best, q, -jnp.inf), axis=1,
                           keepdims=True)
            qb = jnp.maximum(qb, qsel)

        valid = jax.lax.broadcasted_iota(jnp.int32, (_TI, 1), 0) < rows_b
        kept = (hmax >= area_t * (1.0 / 3.0)) & valid
        kl = ent + qb
        terms = w * (_TAU * _TAU) * kl
        return jnp.sum(jnp.where(kept, terms, 0.0))

    sum_add = jax.lax.cond(rows_b > 0, scan_b, lambda: 0.0)

    cnt_ref[...] += jnp.full(cnt_ref.shape, cnt_add, jnp.float32)
    sum_ref[...] += jnp.full(sum_ref.shape, sum_add, jnp.float32)


def kernel(t_boxes, t_logits, t_valid, s_boxes, s_logits, s_valid):
    B, M, C = t_logits.shape
    dt = jnp.float32

    tbp = t_boxes.astype(dt)
    sbp = s_boxes.astype(dt)
    tlp = t_logits.astype(dt)
    slp = s_logits.astype(dt)
    tmf = t_valid.astype(dt)[..., None]
    sbt = sbp.transpose(0, 2, 1)

    fa, fb, cp, lse_col = pl.pallas_call(
        _stats_kernel,
        grid=(B,),
        in_specs=[
            pl.BlockSpec((1, M, 4), lambda i: (i, 0, 0)),
            pl.BlockSpec((1, M, 4), lambda i: (i, 0, 0)),
            pl.BlockSpec((1, M, C), lambda i: (i, 0, 0)),
            pl.BlockSpec((1, M, 1), lambda i: (i, 0, 0)),
            pl.BlockSpec((1, M, C), lambda i: (i, 0, 0)),
        ],
        out_specs=[
            pl.BlockSpec((1, M, 1), lambda i: (i, 0, 0)),
            pl.BlockSpec((1, M, 1), lambda i: (i, 0, 0)),
            pl.BlockSpec((1, 8, 128), lambda i: (i, 0, 0)),
            pl.BlockSpec((1, M, 1), lambda i: (i, 0, 0)),
        ],
        out_shape=[
            jax.ShapeDtypeStruct((B, M, 1), dt),
            jax.ShapeDtypeStruct((B, M, 1), dt),
            jax.ShapeDtypeStruct((B, 8, 128), dt),
            jax.ShapeDtypeStruct((B, M, 1), dt),
        ],
        compiler_params=pltpu.CompilerParams(
            dimension_semantics=("parallel",)),
    )(tbp, sbp, tlp, tmf, slp)

    fa2 = fa[:, :, 0]
    fb2 = fb[:, :, 0]
    idxa = jnp.argsort(-fa2, axis=1).astype(jnp.int32).reshape(-1)
    idxb = jnp.argsort(-fb2, axis=1).astype(jnp.int32).reshape(-1)
    na = jnp.sum(fa2, axis=1).astype(jnp.int32)
    nb = jnp.sum(fb2, axis=1).astype(jnp.int32)

    nt = (M + _TI - 1) // _TI
    grid_spec = pltpu.PrefetchScalarGridSpec(
        num_scalar_prefetch=4,
        grid=(B, nt),
        in_specs=[
            pl.BlockSpec((1, M, 4), lambda i, t, *_: (i, 0, 0)),
            pl.BlockSpec((1, M, C), lambda i, t, *_: (i, 0, 0)),
            pl.BlockSpec((1, 4, M), lambda i, t, *_: (i, 0, 0)),
            pl.BlockSpec((1, M, C), lambda i, t, *_: (i, 0, 0)),
            pl.BlockSpec((1, M, 1), lambda i, t, *_: (i, 0, 0)),
        ],
        out_specs=[
            pl.BlockSpec((1, 8, 128), lambda i, t, *_: (i, 0, 0)),
            pl.BlockSpec((1, 8, 128), lambda i, t, *_: (i, 0, 0)),
        ],
        scratch_shapes=[
            pltpu.VMEM((_TI, 4), dt),
            pltpu.VMEM((_TI, C), dt),
        ],
    )
    cnts, sums = pl.pallas_call(
        functools.partial(_scan_kernel, m=M),
        grid_spec=grid_spec,
        out_shape=[
            jax.ShapeDtypeStruct((B, 8, 128), dt),
            jax.ShapeDtypeStruct((B, 8, 128), dt),
        ],
        compiler_params=pltpu.CompilerParams(
            dimension_semantics=("parallel", "arbitrary")),
    )(idxa, na, idxb, nb, tbp, tlp, sbt, slp, lse_col)

    s = sums[:, 0, 0]
    n = cp[:, 0, 0] + cnts[:, 0, 0]
    has = n > 0
    loss_i = jnp.where(has, s / jnp.maximum(n, 1.0), 0.0)
    loss_sum = jnp.sum(loss_i)
    denom = jnp.sum(has.astype(dt))
    return jnp.where(denom == 0, loss_sum, loss_sum / jnp.maximum(denom, 1.0))
